# baseline (speedup 1.0000x reference)
import math

import jax
import jax.numpy as jnp
from jax import lax
from jax.experimental import pallas as pl
from jax.experimental.pallas import tpu as pltpu
from jax.experimental.pallas import tpu_sc as plsc

D_MODEL = 64
SCALE = math.sqrt(D_MODEL)
NC, NS = 2, 16
NW = NC * NS
LANES = 16
CHUNK = 128


def _embed_body(x_hbm, lut_hbm, out_hbm, idx_v, rows_v, sem):
    wid = lax.axis_index("s") * NC + lax.axis_index("c")
    b_per_w = x_hbm.shape[0] // NW
    n_chunks = b_per_w // CHUNK
    base = wid * b_per_w

    def chunk_body(g, carry):
        start = base + g * CHUNK
        pltpu.sync_copy(x_hbm.at[pl.ds(start, CHUNK)], idx_v)
        pltpu.async_copy(lut_hbm.at[idx_v], rows_v, sem).wait()

        def scale_row(i, c):
            for j in range(D_MODEL // LANES):
                sl = pl.ds(j * LANES, LANES)
                rows_v[i, sl] = rows_v[i, sl] * SCALE
            return c

        lax.fori_loop(0, CHUNK, scale_row, 0)
        pltpu.sync_copy(rows_v, out_hbm.at[pl.ds(start, CHUNK), :])
        return carry

    lax.fori_loop(0, n_chunks, chunk_body, 0)


def kernel(x, lut):
    B = x.shape[0] * x.shape[1]
    xf = x.reshape(B)
    k = pl.kernel(
        _embed_body,
        out_type=jax.ShapeDtypeStruct((B, D_MODEL), jnp.float32),
        mesh=plsc.VectorSubcoreMesh(core_axis_name="c", subcore_axis_name="s"),
        scratch_types=[
            pltpu.VMEM((CHUNK,), jnp.int32),
            pltpu.VMEM((CHUNK, D_MODEL), jnp.float32),
            pltpu.SemaphoreType.DMA,
        ],
        compiler_params=pltpu.CompilerParams(use_tc_tiling_on_sc=False),
    )
    out = k(xf, lut)
    return out.reshape(x.shape + (D_MODEL,))


# speedup vs baseline: 1.2775x; 1.2775x over previous
import math

import jax
import jax.numpy as jnp
from jax import lax
from jax.experimental import pallas as pl
from jax.experimental.pallas import tpu as pltpu
from jax.experimental.pallas import tpu_sc as plsc

D_MODEL = 64
SCALE = math.sqrt(D_MODEL)
NC, NS = 2, 16
NW = NC * NS
LANES = 16
CHUNK = 128
NBUF = 4


def _embed_body(x_hbm, lut_hbm, out_hbm, idx_v, grows, wrows, gsem, wsem):
    wid = lax.axis_index("s") * NC + lax.axis_index("c")
    n_chunks = x_hbm.shape[1]
    base = wid * n_chunks * CHUNK

    pltpu.sync_copy(x_hbm.at[wid], idx_v)

    for b in range(NBUF):
        pltpu.async_copy(lut_hbm.at[idx_v.at[b]], grows.at[b], gsem.at[b])

    def outer(t_idx, carry):
        t = t_idx * NBUF
        for b in range(NBUF):
            g = t + b
            pltpu.make_async_copy(
                lut_hbm.at[idx_v.at[b]], grows.at[b], gsem.at[b]).wait()

            @pl.when(g >= NBUF)
            def _():
                pltpu.make_async_copy(
                    wrows.at[b], out_hbm.at[pl.ds(0, CHUNK), :],
                    wsem.at[b]).wait()

            @plsc.parallel_loop(0, CHUNK, step=2, unroll=2)
            def _(i):
                for r in range(2):
                    for j in range(D_MODEL // LANES):
                        sl = pl.ds(j * LANES, LANES)
                        wrows[b, i + r, sl] = grows[b, i + r, sl] * SCALE

            pltpu.async_copy(
                wrows.at[b], out_hbm.at[pl.ds(base + g * CHUNK, CHUNK), :],
                wsem.at[b])

            @pl.when(g + NBUF < n_chunks)
            def _():
                pltpu.async_copy(
                    lut_hbm.at[idx_v.at[g + NBUF]], grows.at[b], gsem.at[b])

        return carry

    lax.fori_loop(0, n_chunks // NBUF, outer, 0)

    for b in range(NBUF):
        pltpu.make_async_copy(
            wrows.at[b], out_hbm.at[pl.ds(0, CHUNK), :], wsem.at[b]).wait()


def kernel(x, lut):
    B = x.shape[0] * x.shape[1]
    n_chunks = B // (NW * CHUNK)
    xf = x.reshape(NW, n_chunks, CHUNK)
    k = pl.kernel(
        _embed_body,
        out_type=jax.ShapeDtypeStruct((B, D_MODEL), jnp.float32),
        mesh=plsc.VectorSubcoreMesh(core_axis_name="c", subcore_axis_name="s"),
        scratch_types=[
            pltpu.VMEM((n_chunks, CHUNK), jnp.int32),
            pltpu.VMEM((NBUF, CHUNK, D_MODEL), jnp.float32),
            pltpu.VMEM((NBUF, CHUNK, D_MODEL), jnp.float32),
            pltpu.SemaphoreType.DMA((NBUF,)),
            pltpu.SemaphoreType.DMA((NBUF,)),
        ],
        compiler_params=pltpu.CompilerParams(use_tc_tiling_on_sc=False),
    )
    out = k(xf, lut)
    return out.reshape(x.shape + (D_MODEL,))
